# baseline (speedup 1.0000x reference)
import functools

import jax
import jax.numpy as jnp
from jax import lax
from jax.experimental import pallas as pl
from jax.experimental.pallas import tpu as pltpu
from jax.experimental.pallas import tpu_sc as plsc

_NC, _NS, _L = 2, 16, 16
_NWK = _NC * _NS
_BS, _SEQ, _FEAT, _OUT = 8, 2048, 1024, 1024
_FPT = _FEAT // _NWK
_HROWS = _SEQ // 2
_NST = _BS * 2


def _sc_pool(feat_hbm, score_hbm, idx_hbm, zero_hbm, out_hbm,
             buf0, buf1, idxb, scb, acc0, acc1, sem):
    c = lax.axis_index("c")
    s = lax.axis_index("s")
    w = s * _NC + c
    f0 = w * _FPT

    col = lax.iota(jnp.int32, _L)
    bufs = (buf0, buf1)

    def _feat_copy(st, buf):
        b, h = divmod(st, 2)
        return pltpu.make_async_copy(
            feat_hbm.at[b, pl.ds(h * _HROWS, _HROWS), pl.ds(f0, _FPT)],
            buf,
            sem,
        )

    _feat_copy(0, buf0).start()
    for st in range(_NST):
        b, h = divmod(st, 2)
        if st + 1 < _NST:
            _feat_copy(st + 1, bufs[(st + 1) % 2]).start()
        if h == 0:
            pltpu.sync_copy(zero_hbm, acc0)
            pltpu.sync_copy(zero_hbm, acc1)
            pltpu.sync_copy(idx_hbm.at[b], idxb)
            pltpu.sync_copy(score_hbm.at[b], scb)
        buf = bufs[st % 2]
        _feat_copy(st, buf).wait()

        def _grp(g, gi, h=h, buf=buf):
            iv = idxb[pl.ds(h * _HROWS + g * _L, _L)]
            sv = scb[pl.ds(h * _HROWS + g * _L, _L)]
            for r in range(_L):
                row = g * _L + r
                rowids = jnp.full((_L,), iv[r])
                bc = jnp.full((_L,), sv[r])
                v0 = buf[row, pl.ds(0, _L)] * bc
                plsc.addupdate_scatter(acc0, [rowids, col], v0)
                v1 = buf[row, pl.ds(_L, _L)] * bc
                plsc.addupdate_scatter(acc1, [rowids, col], v1)
            return gi

        lax.fori_loop(0, _HROWS // _L, _grp, 0)

        if h == 1:
            pltpu.sync_copy(acc0, out_hbm.at[b, :, pl.ds(f0, _L)])
            pltpu.sync_copy(acc1, out_hbm.at[b, :, pl.ds(f0 + _L, _L)])


_sc_call = functools.partial(
    pl.kernel,
    out_type=jax.ShapeDtypeStruct((_BS, _OUT, _FEAT), jnp.float32),
    mesh=plsc.VectorSubcoreMesh(core_axis_name="c", subcore_axis_name="s"),
    compiler_params=pltpu.CompilerParams(
        use_tc_tiling_on_sc=False, needs_layout_passes=False
    ),
    scratch_types=[
        pltpu.VMEM((_HROWS, _FPT), jnp.float32),
        pltpu.VMEM((_HROWS, _FPT), jnp.float32),
        pltpu.VMEM((_SEQ,), jnp.int32),
        pltpu.VMEM((_SEQ,), jnp.float32),
        pltpu.VMEM((_OUT, _L), jnp.float32),
        pltpu.VMEM((_OUT, _L), jnp.float32),
        pltpu.SemaphoreType.DMA,
    ],
)(_sc_pool)


def kernel(score, feature, out_len):
    s2 = score[:, :, 0]

    cumsum = jnp.cumsum(score, axis=1)
    cumsum = jnp.where(jnp.mod(cumsum, 1.0) < 0.01, cumsum - 0.01, cumsum)
    int_cumsum = jnp.floor(cumsum).astype(jnp.int32)
    int_cumsum = jnp.clip(int_cumsum, 0, out_len - 1)
    idx = int_cumsum[:, :, 0]

    zeros = jnp.zeros((_OUT, _L), jnp.float32)
    return _sc_call(feature, s2, idx, zeros)


# speedup vs baseline: 2.2163x; 2.2163x over previous
import functools

import jax
import jax.numpy as jnp
from jax import lax
from jax.experimental import pallas as pl
from jax.experimental.pallas import tpu as pltpu
from jax.experimental.pallas import tpu_sc as plsc

_BS, _SEQ, _FEAT, _OUT = 8, 2048, 1024, 1024

_BTC = _BS - 1
_WIN = 256
_NWIN = _SEQ // _WIN
_SPAN = 272

_NC, _NS, _L = 2, 16, 16
_NWK = _NC * _NS
_FPT = _FEAT // _NWK
_HROWS = _SEQ // 2


def _wpool_kernel(idx_ref, score_ref, feat_ref, out_ref):
    w = pl.program_id(1)

    @pl.when(w == 0)
    def _():
        out_ref[...] = jnp.zeros_like(out_ref)

    v0 = idx_ref[0, 0, 0, 0]
    j0 = jnp.minimum((v0 // 8) * 8, _OUT - _SPAN)
    j0 = pl.multiple_of(j0, 8)
    rows = jax.lax.broadcasted_iota(jnp.int32, (_SPAN, 1), 0) + j0
    a = jnp.where(idx_ref[0, 0] == rows, score_ref[0, 0], 0.0)
    part = jax.lax.dot(a, feat_ref[0], preferred_element_type=jnp.float32)
    out_ref[0, pl.ds(j0, _SPAN), :] += part


def _sc_pool(feat_hbm, score_hbm, idx_hbm, zero_hbm, out_hbm,
             buf0, buf1, idxb, scb, acc0, acc1, sem):
    c = lax.axis_index("c")
    s = lax.axis_index("s")
    w = s * _NC + c
    f0 = w * _FPT

    col = lax.iota(jnp.int32, _L)
    bufs = (buf0, buf1)

    def _feat_copy(h, buf):
        return pltpu.make_async_copy(
            feat_hbm.at[0, pl.ds(h * _HROWS, _HROWS), pl.ds(f0, _FPT)],
            buf,
            sem,
        )

    _feat_copy(0, buf0).start()
    pltpu.sync_copy(zero_hbm, acc0)
    pltpu.sync_copy(zero_hbm, acc1)
    pltpu.sync_copy(idx_hbm.at[0], idxb)
    pltpu.sync_copy(score_hbm.at[0], scb)
    for h in range(2):
        if h + 1 < 2:
            _feat_copy(h + 1, bufs[(h + 1) % 2]).start()
        buf = bufs[h % 2]
        _feat_copy(h, buf).wait()

        def _grp(g, gi, h=h, buf=buf):
            iv = idxb[pl.ds(h * _HROWS + g * _L, _L)]
            sv = scb[pl.ds(h * _HROWS + g * _L, _L)]
            for r in range(_L):
                row = g * _L + r
                rowids = jnp.full((_L,), iv[r])
                bc = jnp.full((_L,), sv[r])
                v0 = buf[row, pl.ds(0, _L)] * bc
                plsc.addupdate_scatter(acc0, [rowids, col], v0)
                v1 = buf[row, pl.ds(_L, _L)] * bc
                plsc.addupdate_scatter(acc1, [rowids, col], v1)
            return gi

        lax.fori_loop(0, _HROWS // _L, _grp, 0)

    pltpu.sync_copy(acc0, out_hbm.at[0, :, pl.ds(f0, _L)])
    pltpu.sync_copy(acc1, out_hbm.at[0, :, pl.ds(f0 + _L, _L)])


_sc_call = functools.partial(
    pl.kernel,
    out_type=jax.ShapeDtypeStruct((1, _OUT, _FEAT), jnp.float32),
    mesh=plsc.VectorSubcoreMesh(core_axis_name="c", subcore_axis_name="s"),
    compiler_params=pltpu.CompilerParams(
        use_tc_tiling_on_sc=False, needs_layout_passes=False
    ),
    scratch_types=[
        pltpu.VMEM((_HROWS, _FPT), jnp.float32),
        pltpu.VMEM((_HROWS, _FPT), jnp.float32),
        pltpu.VMEM((_SEQ,), jnp.int32),
        pltpu.VMEM((_SEQ,), jnp.float32),
        pltpu.VMEM((_OUT, _L), jnp.float32),
        pltpu.VMEM((_OUT, _L), jnp.float32),
        pltpu.SemaphoreType.DMA,
    ],
)(_sc_pool)


def kernel(score, feature, out_len):
    s2 = score[:, :, 0]

    cumsum = jnp.cumsum(score, axis=1)
    cumsum = jnp.where(jnp.mod(cumsum, 1.0) < 0.01, cumsum - 0.01, cumsum)
    int_cumsum = jnp.floor(cumsum).astype(jnp.int32)
    int_cumsum = jnp.clip(int_cumsum, 0, out_len - 1)
    idx = int_cumsum[:, :, 0]

    zeros = jnp.zeros((_OUT, _L), jnp.float32)
    out_sc = _sc_call(
        feature[_BTC:], s2[_BTC:], idx[_BTC:], zeros
    )

    idx4 = idx[:_BTC].reshape(_BTC, _NWIN, 1, _WIN)
    s4 = s2[:_BTC].reshape(_BTC, _NWIN, 1, _WIN)
    out_tc = pl.pallas_call(
        _wpool_kernel,
        grid=(_BTC, _NWIN),
        in_specs=[
            pl.BlockSpec((1, 1, 1, _WIN), lambda b, w: (b, w, 0, 0)),
            pl.BlockSpec((1, 1, 1, _WIN), lambda b, w: (b, w, 0, 0)),
            pl.BlockSpec((1, _WIN, _FEAT), lambda b, w: (b, w, 0)),
        ],
        out_specs=pl.BlockSpec((1, _OUT, _FEAT), lambda b, w: (b, 0, 0)),
        out_shape=jax.ShapeDtypeStruct((_BTC, _OUT, _FEAT), jnp.float32),
    )(idx4, s4, feature[:_BTC])

    return jnp.concatenate([out_tc, out_sc], axis=0)


# speedup vs baseline: 6.2332x; 2.8124x over previous
import jax
import jax.numpy as jnp
from jax.experimental import pallas as pl

_BS = 8
_SEQ = 2048
_FEAT = 1024
_OUT = 1024
_WIN = 512
_NW = _SEQ // _WIN
_SPAN = 528


def _wpool_kernel(idx_ref, score_ref, feat_ref, out_ref):
    w = pl.program_id(1)

    @pl.when(w == 0)
    def _():
        out_ref[...] = jnp.zeros_like(out_ref)

    v0 = idx_ref[0, 0, 0, 0]
    j0 = jnp.minimum((v0 // 8) * 8, _OUT - _SPAN)
    j0 = pl.multiple_of(j0, 8)
    rows = jax.lax.broadcasted_iota(jnp.int32, (_SPAN, 1), 0) + j0
    a = jnp.where(idx_ref[0, 0] == rows, score_ref[0, 0], 0.0)
    part = jax.lax.dot(a, feat_ref[0], preferred_element_type=jnp.float32)
    out_ref[0, pl.ds(j0, _SPAN), :] += part


def kernel(score, feature, out_len):
    s2 = score[:, :, 0]

    cumsum = jnp.cumsum(score, axis=1)
    cumsum = jnp.where(jnp.mod(cumsum, 1.0) < 0.01, cumsum - 0.01, cumsum)
    int_cumsum = jnp.floor(cumsum).astype(jnp.int32)
    int_cumsum = jnp.clip(int_cumsum, 0, out_len - 1)
    idx = int_cumsum[:, :, 0]

    idx4 = idx.reshape(_BS, _NW, 1, _WIN)
    s4 = s2.reshape(_BS, _NW, 1, _WIN)

    out = pl.pallas_call(
        _wpool_kernel,
        grid=(_BS, _NW),
        in_specs=[
            pl.BlockSpec((1, 1, 1, _WIN), lambda b, w: (b, w, 0, 0)),
            pl.BlockSpec((1, 1, 1, _WIN), lambda b, w: (b, w, 0, 0)),
            pl.BlockSpec((1, _WIN, _FEAT), lambda b, w: (b, w, 0)),
        ],
        out_specs=pl.BlockSpec((1, _OUT, _FEAT), lambda b, w: (b, 0, 0)),
        out_shape=jax.ShapeDtypeStruct((_BS, _OUT, _FEAT), jnp.float32),
    )(idx4, s4, feature)
    return out
